# baseline (speedup 1.0000x reference)
import functools

import jax
import jax.numpy as jnp
from jax import lax
from jax.experimental import pallas as pl
from jax.experimental.pallas import tpu as pltpu
from jax.experimental.pallas import tpu_sc as plsc

BATCH = 16384
LANES = 16


def _make_kernel(num_cores, num_subcores):
    nw = num_cores * num_subcores
    pairs_per_w = BATCH // nw
    ids_per_w = 2 * pairs_per_w
    rows = ids_per_w // 128
    groups = pairs_per_w // LANES

    mesh = plsc.VectorSubcoreMesh(core_axis_name="c", subcore_axis_name="s")

    @functools.partial(
        pl.kernel,
        mesh=mesh,
        out_type=jax.ShapeDtypeStruct((BATCH,), jnp.float32),
        scratch_types=[
            pltpu.VMEM((rows, 128), jnp.int32),
            pltpu.VMEM((rows, 128), jnp.float32),
            pltpu.VMEM((pairs_per_w,), jnp.float32),
            pltpu.SemaphoreType.DMA,
        ],
        compiler_params=pltpu.CompilerParams(
            needs_layout_passes=False,
            disable_bounds_checks=True,
            disable_semaphore_checks=True,
        ),
    )
    def bt_kernel(xr_hbm, s_hbm, out_hbm, idx_v, val_v, out_v, sem):
        wid = lax.axis_index("s") * num_cores + lax.axis_index("c")
        pltpu.sync_copy(xr_hbm.at[wid], idx_v)
        copies = [
            pltpu.async_copy(s_hbm.at[idx_v.at[j]], val_v.at[j], sem)
            for j in range(rows)
        ]
        for c in copies:
            c.wait()
        lanes = lax.iota(jnp.int32, LANES)
        for i in range(groups):
            r = i // 4
            cols = 32 * (i % 4) + 2 * lanes
            row_idx = jnp.full((LANES,), r, jnp.int32)
            s_w = plsc.load_gather(val_v, [row_idx, cols])
            s_l = plsc.load_gather(val_v, [row_idx, cols + 1])
            out_v[pl.ds(i * LANES, LANES)] = s_w / (s_w + s_l)
        pltpu.sync_copy(out_v, out_hbm.at[pl.ds(wid * pairs_per_w, pairs_per_w)])

    return bt_kernel


def kernel(x, strengths):
    info = plsc.get_sparse_core_info()
    fn = _make_kernel(info.num_cores, info.num_subcores)
    nw = info.num_cores * info.num_subcores
    xr = x.astype(jnp.int32).reshape(nw, (2 * BATCH) // (nw * 128), 128)
    return fn(xr, strengths)


# speedup vs baseline: 1.1161x; 1.1161x over previous
import functools

import jax
import jax.numpy as jnp
from jax import lax
from jax.experimental import pallas as pl
from jax.experimental.pallas import tpu as pltpu
from jax.experimental.pallas import tpu_sc as plsc

BATCH = 16384
LANES = 16


def _make_kernel(num_cores, num_subcores):
    nw = num_cores * num_subcores
    pairs_per_w = BATCH // nw
    ids_per_w = 2 * pairs_per_w
    rows = ids_per_w // 128
    groups = pairs_per_w // LANES

    mesh = plsc.VectorSubcoreMesh(core_axis_name="c", subcore_axis_name="s")

    @functools.partial(
        pl.kernel,
        mesh=mesh,
        out_type=jax.ShapeDtypeStruct((BATCH,), jnp.float32),
        scratch_types=[
            pltpu.VMEM((pairs_per_w, 2), jnp.int32),
            pltpu.VMEM((rows, 128), jnp.int32),
            pltpu.VMEM((rows, 128), jnp.float32),
            pltpu.VMEM((pairs_per_w,), jnp.float32),
            pltpu.SemaphoreType.DMA,
        ],
        compiler_params=pltpu.CompilerParams(
            needs_layout_passes=False,
            disable_bounds_checks=True,
            disable_semaphore_checks=True,
        ),
    )
    def bt_kernel(x_hbm, s_hbm, out_hbm, xv, idx_v, val_v, out_v, sem):
        wid = lax.axis_index("s") * num_cores + lax.axis_index("c")
        pltpu.sync_copy(x_hbm.at[pl.ds(wid * pairs_per_w, pairs_per_w), :], xv)
        lanes = lax.iota(jnp.int32, LANES)
        for q in range(rows):
            for t in range(8):
                base = q * 128 + t * LANES
                flat = base + lanes
                idx_v[q, pl.ds(t * LANES, LANES)] = plsc.load_gather(
                    xv, [flat // 2, flat % 2]
                )
        copies = [
            pltpu.async_copy(s_hbm.at[idx_v.at[j]], val_v.at[j], sem)
            for j in range(rows)
        ]
        for c in copies:
            c.wait()
        for i in range(groups):
            r = i // 4
            cols = 32 * (i % 4) + 2 * lanes
            row_idx = jnp.full((LANES,), r, jnp.int32)
            s_w = plsc.load_gather(val_v, [row_idx, cols])
            s_l = plsc.load_gather(val_v, [row_idx, cols + 1])
            out_v[pl.ds(i * LANES, LANES)] = s_w / (s_w + s_l)
        pltpu.sync_copy(out_v, out_hbm.at[pl.ds(wid * pairs_per_w, pairs_per_w)])

    return bt_kernel


def kernel(x, strengths):
    info = plsc.get_sparse_core_info()
    fn = _make_kernel(info.num_cores, info.num_subcores)
    return fn(x.astype(jnp.int32), strengths)
